# Initial kernel scaffold; baseline (speedup 1.0000x reference)
import jax
import jax.numpy as jnp
from jax.experimental import pallas as pl


def kernel(x, table, W, b):
    raise NotImplementedError("write your pallas kernel here")


# speedup vs baseline: 9.5840x; 9.5840x over previous
import functools

import jax
import jax.numpy as jnp
from jax import lax
from jax.experimental import pallas as pl
from jax.experimental.pallas import tpu as pltpu
from jax.experimental.pallas import tpu_sc as plsc

BATCH = 4096
SEQ = 200
EMB = 64
HALF = SEQ // 2
NC = 2
NS = 16
NW = NC * NS
BPW = BATCH // NW
NEG_INF = float("-inf")


def _pool_body(table_hbm, x2_hbm, out_hbm, idx_v, buf_v, pooled_v, sem):
    wid = lax.axis_index("s") * NC + lax.axis_index("c")
    base = wid * BPW
    pltpu.sync_copy(x2_hbm.at[pl.ds(2 * base, 2 * BPW)], idx_v)

    def row_step(i, _):
        c0 = pltpu.async_copy(table_hbm.at[idx_v.at[2 * i]], buf_v.at[0], sem)
        c1 = pltpu.async_copy(table_hbm.at[idx_v.at[2 * i + 1]], buf_v.at[1], sem)
        c0.wait()
        c1.wait()

        def max_step(r, accs):
            new = []
            for h in range(2):
                for c in range(4):
                    v = buf_v[h, r, pl.ds(c * 16, 16)]
                    new.append(jnp.maximum(accs[h * 4 + c], v))
            return tuple(new)

        init = tuple(jnp.full((16,), NEG_INF, jnp.float32) for _ in range(8))
        accs = lax.fori_loop(0, HALF, max_step, init)
        for c in range(4):
            pooled_v[i, pl.ds(c * 16, 16)] = jnp.maximum(accs[c], accs[4 + c])
        return ()

    lax.fori_loop(0, BPW, row_step, ())
    pltpu.sync_copy(pooled_v, out_hbm.at[pl.ds(base, BPW)])


def _sc_pool(x2, table):
    mesh = plsc.VectorSubcoreMesh(
        core_axis_name="c", subcore_axis_name="s", num_cores=NC, num_subcores=NS
    )
    fn = pl.kernel(
        _pool_body,
        out_type=jax.ShapeDtypeStruct((BATCH, EMB), jnp.float32),
        mesh=mesh,
        scratch_types=[
            pltpu.VMEM((2 * BPW, HALF), jnp.int32),
            pltpu.VMEM((2, HALF, EMB), jnp.float32),
            pltpu.VMEM((BPW, EMB), jnp.float32),
            pltpu.SemaphoreType.DMA,
        ],
        compiler_params=pltpu.CompilerParams(use_tc_tiling_on_sc=False),
    )
    return fn(table, x2)


def _linear_body(p_ref, w_ref, b_ref, o_ref):
    o_ref[...] = (
        lax.dot_general(
            p_ref[...], w_ref[...], (((1,), (1,)), ((), ())),
            preferred_element_type=jnp.float32,
        )
        + b_ref[...]
    )


def _tc_linear(pooled, W, b):
    return pl.pallas_call(
        _linear_body,
        out_shape=jax.ShapeDtypeStruct((BATCH, 2), jnp.float32),
    )(pooled, W, b.reshape(1, 2))


@jax.jit
def kernel(x, table, W, b):
    x2 = x.astype(jnp.int32).reshape(2 * BATCH, HALF)
    pooled = _sc_pool(x2, table)
    return _tc_linear(pooled, W, b)


# speedup vs baseline: 14.1903x; 1.4806x over previous
import functools

import jax
import jax.numpy as jnp
from jax import lax
from jax.experimental import pallas as pl
from jax.experimental.pallas import tpu as pltpu
from jax.experimental.pallas import tpu_sc as plsc

BATCH = 4096
SEQ = 200
EMB = 64
HALF = SEQ // 2
NC = 2
NS = 16
NW = NC * NS
BPW = BATCH // NW
NEG_INF = float("-inf")


def _pool_body(table_hbm, x2_hbm, out_hbm, idx_v, buf_v, pooled_v, sem0, sem1):
    wid = lax.axis_index("s") * NC + lax.axis_index("c")
    base = wid * BPW
    sems = (sem0, sem1)
    pltpu.sync_copy(x2_hbm.at[pl.ds(2 * base, 2 * BPW)], idx_v)

    def fire(row, st):
        for h in range(2):
            pltpu.async_copy(
                table_hbm.at[idx_v.at[2 * row + h]], buf_v.at[st, h], sems[st]
            )

    def drain(st):
        for h in range(2):
            pltpu.make_async_copy(
                table_hbm.at[idx_v.at[h]], buf_v.at[st, h], sems[st]
            ).wait()

    def compute(row, st):
        def max_step(r, accs):
            new = list(accs)
            for u in range(2):
                for h in range(2):
                    for c in range(4):
                        v = buf_v[st, h, 2 * r + u, pl.ds(c * 16, 16)]
                        k = h * 4 + c
                        new[k] = jnp.maximum(new[k], v)
            return tuple(new)

        init = tuple(jnp.full((16,), NEG_INF, jnp.float32) for _ in range(8))
        accs = lax.fori_loop(0, HALF // 2, max_step, init)
        for c in range(4):
            pooled_v[row, pl.ds(c * 16, 16)] = jnp.maximum(accs[c], accs[4 + c])

    fire(0, 0)

    def pair_step(j, _):
        row0 = 2 * j
        fire(row0 + 1, 1)
        drain(0)
        compute(row0, 0)
        @pl.when(row0 + 2 < BPW)
        def _():
            fire(row0 + 2, 0)

        drain(1)
        compute(row0 + 1, 1)
        return ()

    lax.fori_loop(0, BPW // 2, pair_step, ())
    pltpu.sync_copy(pooled_v, out_hbm.at[pl.ds(base, BPW)])


def _sc_pool(x2, table):
    mesh = plsc.VectorSubcoreMesh(
        core_axis_name="c", subcore_axis_name="s", num_cores=NC, num_subcores=NS
    )
    fn = pl.kernel(
        _pool_body,
        out_type=jax.ShapeDtypeStruct((BATCH, EMB), jnp.float32),
        mesh=mesh,
        scratch_types=[
            pltpu.VMEM((2 * BPW, HALF), jnp.int32),
            pltpu.VMEM((2, 2, HALF, EMB), jnp.float32),
            pltpu.VMEM((BPW, EMB), jnp.float32),
            pltpu.SemaphoreType.DMA,
            pltpu.SemaphoreType.DMA,
        ],
        compiler_params=pltpu.CompilerParams(use_tc_tiling_on_sc=False),
    )
    return fn(table, x2)


def _linear_body(p_ref, w_ref, b_ref, o_ref):
    o_ref[...] = (
        lax.dot_general(
            p_ref[...], w_ref[...], (((1,), (1,)), ((), ())),
            preferred_element_type=jnp.float32,
        )
        + b_ref[...]
    )


def _tc_linear(pooled, W, b):
    return pl.pallas_call(
        _linear_body,
        out_shape=jax.ShapeDtypeStruct((BATCH, 2), jnp.float32),
    )(pooled, W, b.reshape(1, 2))


@jax.jit
def kernel(x, table, W, b):
    x2 = x.astype(jnp.int32).reshape(2 * BATCH, HALF)
    pooled = _sc_pool(x2, table)
    return _tc_linear(pooled, W, b)


# speedup vs baseline: 14.3651x; 1.0123x over previous
import functools

import jax
import jax.numpy as jnp
from jax import lax
from jax.experimental import pallas as pl
from jax.experimental.pallas import tpu as pltpu
from jax.experimental.pallas import tpu_sc as plsc

BATCH = 4096
SEQ = 200
EMB = 64
HALF = SEQ // 2
NC = 2
NS = 16
NW = NC * NS
BPW = BATCH // NW
NEG_INF = float("-inf")


def _pool_body(table_hbm, x2_hbm, out_hbm, idx_v, buf_v, pooled_v, sem0, sem1):
    wid = lax.axis_index("s") * NC + lax.axis_index("c")
    base = wid * BPW
    sems = (sem0, sem1)
    pltpu.sync_copy(x2_hbm.at[pl.ds(base, BPW)], idx_v)

    splits = ((0, 104), (104, 96))

    def fire(row, st):
        for off, ln in splits:
            pltpu.async_copy(
                table_hbm.at[idx_v.at[row, pl.ds(off, ln)]],
                buf_v.at[st, pl.ds(off, ln)],
                sems[st],
            )

    def drain(st):
        for off, ln in splits:
            pltpu.make_async_copy(
                table_hbm.at[idx_v.at[0, pl.ds(off, ln)]],
                buf_v.at[st, pl.ds(off, ln)],
                sems[st],
            ).wait()

    def compute(row, st):
        def max_step(r, accs):
            new = list(accs)
            for u in range(2):
                for c in range(4):
                    v = buf_v[st, 2 * r + u, pl.ds(c * 16, 16)]
                    k = u * 4 + c
                    new[k] = jnp.maximum(new[k], v)
            return tuple(new)

        init = tuple(jnp.full((16,), NEG_INF, jnp.float32) for _ in range(8))
        accs = lax.fori_loop(0, SEQ // 2, max_step, init)
        for c in range(4):
            pooled_v[row, pl.ds(c * 16, 16)] = jnp.maximum(accs[c], accs[4 + c])

    fire(0, 0)

    def pair_step(j, _):
        row0 = 2 * j
        fire(row0 + 1, 1)
        drain(0)
        compute(row0, 0)
        @pl.when(row0 + 2 < BPW)
        def _():
            fire(row0 + 2, 0)

        drain(1)
        compute(row0 + 1, 1)
        return ()

    lax.fori_loop(0, BPW // 2, pair_step, ())
    pltpu.sync_copy(pooled_v, out_hbm.at[pl.ds(base, BPW)])


def _sc_pool(x2, table):
    mesh = plsc.VectorSubcoreMesh(
        core_axis_name="c", subcore_axis_name="s", num_cores=NC, num_subcores=NS
    )
    fn = pl.kernel(
        _pool_body,
        out_type=jax.ShapeDtypeStruct((BATCH, EMB), jnp.float32),
        mesh=mesh,
        scratch_types=[
            pltpu.VMEM((BPW, SEQ), jnp.int32),
            pltpu.VMEM((2, SEQ, EMB), jnp.float32),
            pltpu.VMEM((BPW, EMB), jnp.float32),
            pltpu.SemaphoreType.DMA,
            pltpu.SemaphoreType.DMA,
        ],
        compiler_params=pltpu.CompilerParams(use_tc_tiling_on_sc=False),
    )
    return fn(table, x2)


def _linear_body(p_ref, w_ref, b_ref, o_ref):
    o_ref[...] = (
        lax.dot_general(
            p_ref[...], w_ref[...], (((1,), (1,)), ((), ())),
            preferred_element_type=jnp.float32,
        )
        + b_ref[...]
    )


def _tc_linear(pooled, W, b):
    return pl.pallas_call(
        _linear_body,
        out_shape=jax.ShapeDtypeStruct((BATCH, 2), jnp.float32),
    )(pooled, W, b.reshape(1, 2))


@jax.jit
def kernel(x, table, W, b):
    pooled = _sc_pool(x.astype(jnp.int32), table)
    return _tc_linear(pooled, W, b)


# speedup vs baseline: 14.8349x; 1.0327x over previous
import functools

import jax
import jax.numpy as jnp
from jax import lax
from jax.experimental import pallas as pl
from jax.experimental.pallas import tpu as pltpu
from jax.experimental.pallas import tpu_sc as plsc

VOCAB = 100000
BATCH = 4096
SEQ = 200
EMB = 64
HALF = SEQ // 2
NC = 2
NS = 16
NW = NC * NS
BPW = BATCH // NW
NEG_INF = float("-inf")


def _pool_body(table_hbm, x2_hbm, out_hbm, idx_v, buf_v, pooled_v, sem0, sem1):
    wid = lax.axis_index("s") * NC + lax.axis_index("c")
    base = wid * BPW
    sems = (sem0, sem1)
    pltpu.sync_copy(x2_hbm.at[pl.ds(base, BPW)], idx_v)

    splits = ((0, 104), (104, 96))

    def fire(row, st):
        for off, ln in splits:
            pltpu.async_copy(
                table_hbm.at[idx_v.at[row, pl.ds(off, ln)]],
                buf_v.at[st, pl.ds(off, ln)],
                sems[st],
            )

    def drain(st):
        for off, ln in splits:
            pltpu.make_async_copy(
                table_hbm.at[idx_v.at[0, pl.ds(off, ln)]],
                buf_v.at[st, pl.ds(off, ln)],
                sems[st],
            ).wait()

    def compute(row, st):
        def max_step(r, accs):
            new = list(accs)
            for u in range(2):
                for c in range(4):
                    v = buf_v[st, 2 * r + u, pl.ds(c * 16, 16)]
                    k = u * 4 + c
                    new[k] = jnp.maximum(new[k], v)
            return tuple(new)

        init = tuple(jnp.full((16,), NEG_INF, jnp.float32) for _ in range(8))
        accs = lax.fori_loop(0, SEQ // 2, max_step, init)
        for c in range(4):
            pooled_v[row, pl.ds(c * 16, 16)] = jnp.maximum(accs[c], accs[4 + c])

    fire(0, 0)

    def pair_step(j, _):
        row0 = 2 * j
        fire(row0 + 1, 1)
        drain(0)
        compute(row0, 0)
        @pl.when(row0 + 2 < BPW)
        def _():
            fire(row0 + 2, 0)

        drain(1)
        compute(row0 + 1, 1)
        return ()

    lax.fori_loop(0, BPW // 2, pair_step, ())
    pltpu.sync_copy(pooled_v, out_hbm.at[pl.ds(base, BPW)])


def _sc_pool(x2, table):
    mesh = plsc.VectorSubcoreMesh(
        core_axis_name="c", subcore_axis_name="s", num_cores=NC, num_subcores=NS
    )
    fn = pl.kernel(
        _pool_body,
        out_type=jax.ShapeDtypeStruct((BATCH, EMB), jnp.float32),
        mesh=mesh,
        scratch_types=[
            pltpu.VMEM((BPW, SEQ), jnp.int32),
            pltpu.VMEM((2, SEQ, EMB), jnp.float32),
            pltpu.VMEM((BPW, EMB), jnp.float32),
            pltpu.SemaphoreType.DMA,
            pltpu.SemaphoreType.DMA,
        ],
        compiler_params=pltpu.CompilerParams(use_tc_tiling_on_sc=False),
    )
    return fn(table, x2)


def _linear_body(p_ref, w_ref, b_ref, o_ref):
    o_ref[...] = (
        lax.dot_general(
            p_ref[...], w_ref[...], (((1,), (1,)), ((), ())),
            preferred_element_type=jnp.float32,
        )
        + b_ref[...]
    )


def _tc_linear(pooled, W, b):
    return pl.pallas_call(
        _linear_body,
        out_shape=jax.ShapeDtypeStruct((BATCH, 2), jnp.float32),
    )(pooled, W, b.reshape(1, 2))


@jax.jit
def kernel(x, table, W, b):
    padded = jnp.pad(table, ((0, 0), (0, EMB)))
    view = padded.reshape(2 * VOCAB, EMB)
    xx = x.astype(jnp.int32) * 2
    pooled = _sc_pool(xx, view)
    return _tc_linear(pooled, W, b)


# speedup vs baseline: 18.5741x; 1.2521x over previous
import functools

import jax
import jax.numpy as jnp
from jax import lax
from jax.experimental import pallas as pl
from jax.experimental.pallas import tpu as pltpu
from jax.experimental.pallas import tpu_sc as plsc

VOCAB = 100000
BATCH = 4096
SEQ = 200
EMB = 64
HALF = SEQ // 2
NC = 2
NS = 16
NW = NC * NS
BPW = BATCH // NW
NEG_INF = float("-inf")
NBUF = 4


def _pool_body(table_hbm, x2_hbm, out_hbm, idx_v, buf_v, pooled_v, *sems):
    wid = lax.axis_index("s") * NC + lax.axis_index("c")
    base = wid * BPW
    pltpu.sync_copy(x2_hbm.at[pl.ds(base, BPW)], idx_v)

    splits = ((0, 104), (104, 96))

    def fire(row, st):
        for off, ln in splits:
            pltpu.async_copy(
                table_hbm.at[idx_v.at[row, pl.ds(off, ln)]],
                buf_v.at[st, pl.ds(off, ln)],
                sems[st],
            )

    def drain(st):
        for off, ln in splits:
            pltpu.make_async_copy(
                table_hbm.at[idx_v.at[0, pl.ds(off, ln)]],
                buf_v.at[st, pl.ds(off, ln)],
                sems[st],
            ).wait()

    def compute(row, st):
        def max_step(r, accs):
            new = list(accs)
            for u in range(2):
                for c in range(4):
                    v = buf_v[st, 2 * r + u, pl.ds(c * 16, 16)]
                    k = u * 4 + c
                    new[k] = jnp.maximum(new[k], v)
            return tuple(new)

        init = tuple(jnp.full((16,), NEG_INF, jnp.float32) for _ in range(8))
        accs = lax.fori_loop(0, SEQ // 2, max_step, init)
        for c in range(4):
            pooled_v[row, pl.ds(c * 16, 16)] = jnp.maximum(accs[c], accs[4 + c])

    for st in range(NBUF - 1):
        fire(st, st)

    def ring_step(j, _):
        row0 = NBUF * j
        for st in range(NBUF):
            row = row0 + st
            ahead = row + NBUF - 1

            @pl.when(ahead < BPW)
            def _():
                fire(ahead, (st + NBUF - 1) % NBUF)

            drain(st)
            compute(row, st)
        return ()

    lax.fori_loop(0, BPW // NBUF, ring_step, ())
    pltpu.sync_copy(pooled_v, out_hbm.at[pl.ds(base, BPW)])


def _sc_pool(x2, table):
    mesh = plsc.VectorSubcoreMesh(
        core_axis_name="c", subcore_axis_name="s", num_cores=NC, num_subcores=NS
    )
    fn = pl.kernel(
        _pool_body,
        out_type=jax.ShapeDtypeStruct((BATCH, EMB), jnp.float32),
        mesh=mesh,
        scratch_types=[
            pltpu.VMEM((BPW, SEQ), jnp.int32),
            pltpu.VMEM((NBUF, SEQ, EMB), jnp.float32),
            pltpu.VMEM((BPW, EMB), jnp.float32),
        ]
        + [pltpu.SemaphoreType.DMA] * NBUF,
        compiler_params=pltpu.CompilerParams(use_tc_tiling_on_sc=False),
    )
    return fn(table, x2)


def _linear_body(p_ref, w_ref, b_ref, o_ref):
    o_ref[...] = (
        lax.dot_general(
            p_ref[...], w_ref[...], (((1,), (1,)), ((), ())),
            preferred_element_type=jnp.float32,
        )
        + b_ref[...]
    )


def _tc_linear(pooled, W, b):
    return pl.pallas_call(
        _linear_body,
        out_shape=jax.ShapeDtypeStruct((BATCH, 2), jnp.float32),
    )(pooled, W, b.reshape(1, 2))


@jax.jit
def kernel(x, table, W, b):
    padded = jnp.pad(table, ((0, 0), (0, EMB)))
    view = padded.reshape(2 * VOCAB, EMB)
    xx = x.astype(jnp.int32) * 2
    pooled = _sc_pool(xx, view)
    return _tc_linear(pooled, W, b)
